# Initial kernel scaffold; baseline (speedup 1.0000x reference)
import jax
import jax.numpy as jnp
from jax.experimental import pallas as pl


def kernel(v, dist, dist_emb, edge_index, Wsf_w, Wsf_b, bn1_g, bn1_b, bn2_g, bn2_b, gru_w, gru_b):
    raise NotImplementedError("write your pallas kernel here")


# speedup vs baseline: 1.9639x; 1.9639x over previous
import functools
from math import pi as PI

import jax
import jax.numpy as jnp
from jax import lax
from jax.experimental import pallas as pl
from jax.experimental.pallas import tpu as pltpu
from jax.experimental.pallas import tpu_sc as plsc

N = 10000
E = 320000
H = 128
NG = 50
CUTOFF = 10.0
EPS = 1e-5

NC = 2
NS = 16
NW = NC * NS
CH = 128
NCHUNK = E // CH

_mesh = plsc.VectorSubcoreMesh(core_axis_name="c", subcore_axis_name="s")


@functools.partial(
    pl.kernel,
    out_type=(jax.ShapeDtypeStruct((E, H), jnp.float32),
              jax.ShapeDtypeStruct((E, H), jnp.float32)),
    mesh=_mesh,
    scratch_types=[
        pltpu.VMEM((CH,), jnp.int32),
        pltpu.VMEM((CH,), jnp.int32),
        pltpu.VMEM((CH, H), jnp.float32),
        pltpu.VMEM((CH, H), jnp.float32),
        pltpu.SemaphoreType.DMA,
        pltpu.SemaphoreType.DMA,
    ],
)
def _sc_gather(v_hbm, ii_hbm, jj_hbm, vi_out, vj_out,
               idx_i, idx_j, rows_i, rows_j, sem_i, sem_j):
    wid = lax.axis_index("s") * NC + lax.axis_index("c")
    nloop = (NCHUNK - wid + NW - 1) // NW

    def body(t, carry):
        base = (wid + t * NW) * CH
        pltpu.sync_copy(ii_hbm.at[pl.ds(base, CH)], idx_i)
        pltpu.sync_copy(jj_hbm.at[pl.ds(base, CH)], idx_j)
        ci = pltpu.async_copy(v_hbm.at[idx_i], rows_i, sem_i)
        cj = pltpu.async_copy(v_hbm.at[idx_j], rows_j, sem_j)
        ci.wait()
        cj.wait()
        pltpu.sync_copy(rows_i, vi_out.at[pl.ds(base, CH)])
        pltpu.sync_copy(rows_j, vj_out.at[pl.ds(base, CH)])
        return carry

    lax.fori_loop(0, nloop, body, 0)


NP = 10240


@functools.partial(
    pl.kernel,
    out_type=(jax.ShapeDtypeStruct((NP, H), jnp.float32),
              jax.ShapeDtypeStruct((NP, H), jnp.float32)),
    mesh=_mesh,
    scratch_types=[
        pltpu.VMEM((CH,), jnp.int32),
        pltpu.VMEM((CH, H), jnp.float32),
        pltpu.VMEM_SHARED((NP, H), jnp.float32),
        pltpu.SemaphoreType.DMA,
    ],
)
def _sc_scatter(msg_hbm, ii_hbm, zeros_hbm, out0, out1, idx, rows, acc_sh, sem):
    c = lax.axis_index("c")
    s = lax.axis_index("s")
    wid = s * NC + c
    rpt = NP // NS
    pltpu.sync_copy(zeros_hbm.at[pl.ds(s * rpt, rpt)], acc_sh.at[pl.ds(s * rpt, rpt)])
    plsc.subcore_barrier()
    nloop = (NCHUNK - wid + NW - 1) // NW

    def body(t, carry):
        base = (wid + t * NW) * CH
        pltpu.sync_copy(ii_hbm.at[pl.ds(base, CH)], idx)
        pltpu.sync_copy(msg_hbm.at[pl.ds(base, CH)], rows)
        pltpu.sync_copy(rows, acc_sh.at[idx], add=True)
        return carry

    lax.fori_loop(0, nloop, body, 0)
    plsc.subcore_barrier()

    @pl.when(c == 0)
    def _():
        pltpu.sync_copy(acc_sh.at[pl.ds(s * rpt, rpt)], out0.at[pl.ds(s * rpt, rpt)])

    @pl.when(c == 1)
    def _():
        pltpu.sync_copy(acc_sh.at[pl.ds(s * rpt, rpt)], out1.at[pl.ds(s * rpt, rpt)])


BE = 2000
GRID2 = E // BE


def _mm_body(vi_ref, vj_ref, de_ref, w1_ref, w2_ref, wd_ref, b_ref,
             x_ref, st_ref, acc_ref):
    e = pl.program_id(0)
    x = jnp.dot(vi_ref[...], w1_ref[...], preferred_element_type=jnp.float32)
    x = x + jnp.dot(vj_ref[...], w2_ref[...], preferred_element_type=jnp.float32)
    x = x + jnp.dot(de_ref[...], wd_ref[...], preferred_element_type=jnp.float32)
    x = x + b_ref[...]
    x_ref[...] = x

    @pl.when(e == 0)
    def _():
        acc_ref[...] = jnp.zeros_like(acc_ref)

    acc_ref[...] += jnp.concatenate(
        [jnp.sum(x, 0, keepdims=True), jnp.sum(x * x, 0, keepdims=True)], 0)

    @pl.when(e == GRID2 - 1)
    def _():
        st_ref[...] = acc_ref[...]


_mm = pl.pallas_call(
    _mm_body,
    grid=(GRID2,),
    in_specs=[
        pl.BlockSpec((BE, H), lambda e: (e, 0)),
        pl.BlockSpec((BE, H), lambda e: (e, 0)),
        pl.BlockSpec((BE, NG), lambda e: (e, 0)),
        pl.BlockSpec((H, 2 * H), lambda e: (0, 0)),
        pl.BlockSpec((H, 2 * H), lambda e: (0, 0)),
        pl.BlockSpec((NG, 2 * H), lambda e: (0, 0)),
        pl.BlockSpec((1, 2 * H), lambda e: (0, 0)),
    ],
    out_specs=[
        pl.BlockSpec((BE, 2 * H), lambda e: (e, 0)),
        pl.BlockSpec((2, 2 * H), lambda e: (0, 0)),
    ],
    out_shape=[
        jax.ShapeDtypeStruct((E, 2 * H), jnp.float32),
        jax.ShapeDtypeStruct((2, 2 * H), jnp.float32),
    ],
    scratch_shapes=[pltpu.VMEM((2, 2 * H), jnp.float32)],
)


def _act_body(x_ref, st_ref, g_ref, b_ref, d_ref, msg_ref):
    st = st_ref[...]
    mean = st[0:1, :] * (1.0 / E)
    var = st[1:2, :] * (1.0 / E) - mean * mean
    xn = (x_ref[...] - mean) * lax.rsqrt(var + EPS) * g_ref[...] + b_ref[...]
    cpart = xn[:, :H]
    fpart = xn[:, H:]
    m = jax.nn.softplus(cpart) * jax.nn.sigmoid(fpart)
    cf = 0.5 * (jnp.cos(d_ref[...] * (PI / CUTOFF)) + 1.0)
    msg_ref[...] = m * cf


_act = pl.pallas_call(
    _act_body,
    grid=(GRID2,),
    in_specs=[
        pl.BlockSpec((BE, 2 * H), lambda e: (e, 0)),
        pl.BlockSpec((2, 2 * H), lambda e: (0, 0)),
        pl.BlockSpec((1, 2 * H), lambda e: (0, 0)),
        pl.BlockSpec((1, 2 * H), lambda e: (0, 0)),
        pl.BlockSpec((BE, 1), lambda e: (e, 0)),
    ],
    out_specs=pl.BlockSpec((BE, H), lambda e: (e, 0)),
    out_shape=jax.ShapeDtypeStruct((E, H), jnp.float32),
)


def _fin_body(p0_ref, p1_ref, v_ref, g2_ref, b2_ref, gw1_ref, gw2_ref, gb_ref,
              out_ref):
    x = p0_ref[:N, :] + p1_ref[:N, :]
    mean = jnp.mean(x, 0, keepdims=True)
    var = jnp.mean(x * x, 0, keepdims=True) - mean * mean
    xn = (x - mean) * lax.rsqrt(var + EPS) * g2_ref[...] + b2_ref[...]
    s = jax.nn.sigmoid(
        jnp.dot(v_ref[...], gw1_ref[...], preferred_element_type=jnp.float32)
        + jnp.dot(xn, gw2_ref[...], preferred_element_type=jnp.float32)
        + gb_ref[...])
    out_ref[...] = jax.nn.softplus(s * v_ref[...] + (1.0 - s) * xn)


_fin = pl.pallas_call(
    _fin_body,
    out_shape=jax.ShapeDtypeStruct((N, H), jnp.float32),
)


def kernel(v, dist, dist_emb, edge_index, Wsf_w, Wsf_b,
           bn1_g, bn1_b, bn2_g, bn2_b, gru_w, gru_b):
    jj = edge_index[0].astype(jnp.int32)
    ii = edge_index[1].astype(jnp.int32)
    vi, vj = _sc_gather(v, ii, jj)
    w1 = Wsf_w[:, :H].T
    w2 = Wsf_w[:, H:2 * H].T
    wd = Wsf_w[:, 2 * H:].T
    x, st = _mm(vi, vj, dist_emb, w1, w2, wd, Wsf_b.reshape(1, 2 * H))
    msg = _act(x, st, bn1_g.reshape(1, 2 * H), bn1_b.reshape(1, 2 * H),
               dist.reshape(E, 1))
    p0, p1 = _sc_scatter(msg, ii, jnp.zeros((NP, H), jnp.float32))
    out = _fin(p0, p1, v,
               bn2_g.reshape(1, H), bn2_b.reshape(1, H),
               gru_w[:, :H].T, gru_w[:, H:].T, gru_b.reshape(1, H))
    return out
